# baseline (speedup 1.0000x reference)
import jax
import jax.numpy as jnp
from jax.experimental import pallas as pl


def _sum_k_kernel(x_ref, o_ref):
    x = x_ref[...]
    o_ref[...] = x[0] + x[1] + x[2] + x[3]


def kernel(encoded_edges, encoded_history):
    K, E, d = encoded_edges.shape
    BR = 2000
    grid = (E // BR,)
    return pl.pallas_call(
        _sum_k_kernel,
        grid=grid,
        in_specs=[pl.BlockSpec((K, BR, d), lambda i: (0, i, 0))],
        out_specs=pl.BlockSpec((BR, d), lambda i: (i, 0)),
        out_shape=jax.ShapeDtypeStruct((E, d), encoded_edges.dtype),
    )(encoded_edges)


# speedup vs baseline: 1.0509x; 1.0509x over previous
import jax
import jax.numpy as jnp
from jax.experimental import pallas as pl


def _sum_k_kernel(x_ref, o_ref):
    x = x_ref[...]
    o_ref[...] = x[0] + x[1] + x[2] + x[3]


def kernel(encoded_edges, encoded_history):
    K, E, d = encoded_edges.shape
    BR = 8000
    grid = (E // BR,)
    return pl.pallas_call(
        _sum_k_kernel,
        grid=grid,
        in_specs=[pl.BlockSpec((K, BR, d), lambda i: (0, i, 0))],
        out_specs=pl.BlockSpec((BR, d), lambda i: (i, 0)),
        out_shape=jax.ShapeDtypeStruct((E, d), encoded_edges.dtype),
    )(encoded_edges)
